# baseline (speedup 1.0000x reference)
import functools

import jax
import jax.numpy as jnp
from jax import lax
from jax.experimental import pallas as pl
from jax.experimental.pallas import tpu as pltpu
from jax.experimental.pallas import tpu_sc as plsc

B = 1024
S = 200
H = 128
NW = 32
UNITS_PER_W = B // NW
HALF = S // 2
LANES = 16
NCH = H // LANES
UNROLL = 4
NBUF = 3

_mesh = plsc.VectorSubcoreMesh(core_axis_name="c", subcore_axis_name="s")


@functools.partial(
    pl.kernel,
    mesh=_mesh,
    out_type=jax.ShapeDtypeStruct((B * S, H), jnp.float32),
    scratch_types=[
        pltpu.VMEM((2 * UNITS_PER_W, HALF), jnp.int32),
        pltpu.VMEM((S, H), jnp.float32),
        pltpu.VMEM((S, H), jnp.float32),
        pltpu.VMEM((S, H), jnp.float32),
        pltpu.VMEM((S, H), jnp.float32),
        pltpu.VMEM((H,), jnp.float32),
        pltpu.VMEM((H,), jnp.float32),
        pltpu.SemaphoreType.DMA,
        pltpu.SemaphoreType.DMA,
        pltpu.SemaphoreType.DMA,
        pltpu.SemaphoreType.DMA,
        pltpu.SemaphoreType.DMA,
        pltpu.SemaphoreType.DMA,
    ],
)
def _bert_embed_sc(idx_hbm, tok_hbm, pos_hbm, g_hbm, b_hbm, out_hbm,
                   idx_v, pos_v, buf0, buf1, buf2, gv, bv,
                   gsem0, gsem1, gsem2, ssem0, ssem1, ssem2):
    w = lax.axis_index("s") * 2 + lax.axis_index("c")
    row_base = w * UNITS_PER_W * S

    pltpu.sync_copy(idx_hbm.at[pl.ds(w * (2 * UNITS_PER_W), 2 * UNITS_PER_W)],
                    idx_v)
    pltpu.sync_copy(pos_hbm.at[pl.ds(0, S)], pos_v)
    pltpu.sync_copy(g_hbm, gv)
    pltpu.sync_copy(b_hbm, bv)

    bufs = (buf0, buf1, buf2)
    gsems = (gsem0, gsem1, gsem2)
    ssems = (ssem0, ssem1, ssem2)

    def start_gather(u, k):
        pltpu.async_copy(tok_hbm.at[idx_v.at[2 * u]],
                         bufs[k].at[pl.ds(0, HALF)], gsems[k])
        pltpu.async_copy(tok_hbm.at[idx_v.at[2 * u + 1]],
                         bufs[k].at[pl.ds(HALF, HALF)], gsems[k])

    def wait_gather(u, k):
        pltpu.make_async_copy(tok_hbm.at[idx_v.at[2 * u]],
                              bufs[k].at[pl.ds(0, HALF)], gsems[k]).wait()
        pltpu.make_async_copy(tok_hbm.at[idx_v.at[2 * u + 1]],
                              bufs[k].at[pl.ds(HALF, HALF)], gsems[k]).wait()

    def out_slice(u):
        return out_hbm.at[pl.ds(row_base + u * S, S)]

    def start_scatter(u, k):
        pltpu.async_copy(bufs[k], out_slice(u), ssems[k])

    def wait_scatter(u, k):
        pltpu.make_async_copy(bufs[k], out_slice(u), ssems[k]).wait()

    g_regs = [gv[pl.ds(c * LANES, LANES)] for c in range(NCH)]
    b_regs = [bv[pl.ds(c * LANES, LANES)] for c in range(NCH)]

    _dn = lax.GatherDimensionNumbers(
        offset_dims=(), collapsed_slice_dims=(0,), start_index_map=(0,))
    lane = lax.iota(jnp.int32, LANES)

    def lane_sum(v):
        for shift in (8, 4, 2, 1):
            perm = (lane ^ shift).reshape(LANES, 1)
            v = v + lax.gather(v, perm, _dn, (1,),
                               mode=lax.GatherScatterMode.PROMISE_IN_BOUNDS)
        return v

    def compute(buf):
        def one_row(r):
            e = [buf[r, pl.ds(c * LANES, LANES)] + pos_v[r, pl.ds(c * LANES, LANES)]
                 for c in range(NCH)]

            def tree_sum(vs):
                while len(vs) > 1:
                    vs = [vs[i] + vs[i + 1] for i in range(0, len(vs) - 1, 2)] \
                         + ([vs[-1]] if len(vs) % 2 else [])
                return vs[0]

            s1 = tree_sum(list(e))
            s2 = tree_sum([v * v for v in e])
            mean_v = lane_sum(s1) * jnp.float32(1.0 / H)
            ex2_v = lane_sum(s2) * jnp.float32(1.0 / H)
            var_v = ex2_v - mean_v * mean_v
            x = var_v + jnp.float32(1e-5)
            i = lax.bitcast_convert_type(x, jnp.int32)
            i = jnp.int32(0x5F3759DF) - (i >> 1)
            y = lax.bitcast_convert_type(i, jnp.float32)
            xh = x * jnp.float32(0.5)
            for _ in range(2):
                y = y * (jnp.float32(1.5) - xh * (y * y))
            for c in range(NCH):
                buf[r, pl.ds(c * LANES, LANES)] = (
                    (e[c] - mean_v) * (y * g_regs[c]) + b_regs[c])

        def row(r, carry):
            for uu in range(UNROLL):
                one_row(UNROLL * r + uu)
            return carry
        lax.fori_loop(0, S // UNROLL, row, 0)

    def process(u, k, wait_prev, prefetch, guard_prefetch):
        wait_gather(u, k)
        compute(bufs[k])
        start_scatter(u, k)
        kn = (k + 2) % NBUF
        if wait_prev:
            wait_scatter(u - 1, kn)
        if prefetch:
            if guard_prefetch:
                @pl.when(u + 2 < UNITS_PER_W)
                def _():
                    start_gather(u + 2, kn)
            else:
                start_gather(u + 2, kn)

    start_gather(0, 0)
    start_gather(1, 1)
    process(0, 0, wait_prev=False, prefetch=True, guard_prefetch=False)
    process(1, 1, wait_prev=True, prefetch=True, guard_prefetch=False)

    def outer(g, carry):
        for j in range(NBUF):
            u = 2 + NBUF * g + j
            process(u, (2 + j) % NBUF, wait_prev=True, prefetch=True,
                    guard_prefetch=True)
        return carry
    lax.fori_loop(0, (UNITS_PER_W - 2) // NBUF, outer, 0)

    wait_scatter(UNITS_PER_W - 1, (UNITS_PER_W - 1) % NBUF)


def kernel(indices, token_table, pos_table, gamma, beta):
    idx2 = indices.astype(jnp.int32).reshape(2 * B, HALF)
    out = _bert_embed_sc(idx2, token_table, pos_table, gamma, beta)
    return out.reshape(B, S, H)


# speedup vs baseline: 1.0568x; 1.0568x over previous
import functools

import jax
import jax.numpy as jnp
from jax import lax
from jax.experimental import pallas as pl
from jax.experimental.pallas import tpu as pltpu
from jax.experimental.pallas import tpu_sc as plsc

B = 1024
S = 200
H = 128
NW = 32
UNITS_PER_W = B // NW
HALF = S // 2
LANES = 16
NCH = H // LANES
UNROLL = 4
NBUF = 3

_mesh = plsc.VectorSubcoreMesh(core_axis_name="c", subcore_axis_name="s")


@functools.partial(
    pl.kernel,
    mesh=_mesh,
    out_type=jax.ShapeDtypeStruct((B * S, H), jnp.float32),
    scratch_types=[
        pltpu.VMEM((2 * UNITS_PER_W, HALF), jnp.int32),
        pltpu.VMEM((S, H), jnp.float32),
        pltpu.VMEM((S, H), jnp.float32),
        pltpu.VMEM((S, H), jnp.float32),
        pltpu.VMEM((S, H), jnp.float32),
        pltpu.VMEM((H,), jnp.float32),
        pltpu.VMEM((H,), jnp.float32),
        pltpu.SemaphoreType.DMA,
        pltpu.SemaphoreType.DMA,
        pltpu.SemaphoreType.DMA,
        pltpu.SemaphoreType.DMA,
        pltpu.SemaphoreType.DMA,
        pltpu.SemaphoreType.DMA,
    ],
)
def _bert_embed_sc(idx_hbm, tok_hbm, pos_hbm, g_hbm, b_hbm, out_hbm,
                   idx_v, pos_v, buf0, buf1, buf2, gv, bv,
                   gsem0, gsem1, gsem2, ssem0, ssem1, ssem2):
    w = lax.axis_index("s") * 2 + lax.axis_index("c")
    row_base = w * UNITS_PER_W * S

    bufs = (buf0, buf1, buf2)
    gsems = (gsem0, gsem1, gsem2)
    ssems = (ssem0, ssem1, ssem2)

    pltpu.sync_copy(idx_hbm.at[pl.ds(w * (2 * UNITS_PER_W), 2 * UNITS_PER_W)],
                    idx_v)

    def start_gather(u, k):
        pltpu.async_copy(tok_hbm.at[idx_v.at[2 * u]],
                         bufs[k].at[pl.ds(0, HALF)], gsems[k])
        pltpu.async_copy(tok_hbm.at[idx_v.at[2 * u + 1]],
                         bufs[k].at[pl.ds(HALF, HALF)], gsems[k])

    def wait_gather_half(u, k, h):
        pltpu.make_async_copy(tok_hbm.at[idx_v.at[2 * u + h]],
                              bufs[k].at[pl.ds(h * HALF, HALF)],
                              gsems[k]).wait()

    SC_SPLIT = (0, 96, S)

    def start_scatter_half(u, k, h):
        lo, hi = SC_SPLIT[h], SC_SPLIT[h + 1]
        pltpu.async_copy(bufs[k].at[pl.ds(lo, hi - lo)],
                         out_hbm.at[pl.ds(row_base + u * S + lo, hi - lo)],
                         ssems[k])

    def wait_scatter(u, k):
        pltpu.make_async_copy(bufs[k],
                              out_hbm.at[pl.ds(row_base + u * S, S)],
                              ssems[k]).wait()

    g_regs = [gv[pl.ds(c * LANES, LANES)] for c in range(NCH)]
    b_regs = [bv[pl.ds(c * LANES, LANES)] for c in range(NCH)]

    _dn = lax.GatherDimensionNumbers(
        offset_dims=(), collapsed_slice_dims=(0,), start_index_map=(0,))
    lane = lax.iota(jnp.int32, LANES)

    def lane_sum(v):
        for shift in (8, 4, 2, 1):
            perm = (lane ^ shift).reshape(LANES, 1)
            v = v + lax.gather(v, perm, _dn, (1,),
                               mode=lax.GatherScatterMode.PROMISE_IN_BOUNDS)
        return v

    def compute(buf, half):
        base = half * HALF

        def one_row(r):
            e = [buf[r, pl.ds(c * LANES, LANES)] + pos_v[r, pl.ds(c * LANES, LANES)]
                 for c in range(NCH)]

            def tree_sum(vs):
                while len(vs) > 1:
                    vs = [vs[i] + vs[i + 1] for i in range(0, len(vs) - 1, 2)] \
                         + ([vs[-1]] if len(vs) % 2 else [])
                return vs[0]

            s1 = tree_sum(list(e))
            s2 = tree_sum([v * v for v in e])
            mean_v = lane_sum(s1) * jnp.float32(1.0 / H)
            ex2_v = lane_sum(s2) * jnp.float32(1.0 / H)
            var_v = ex2_v - mean_v * mean_v
            x = var_v + jnp.float32(1e-5)
            i = lax.bitcast_convert_type(x, jnp.int32)
            i = jnp.int32(0x5F3759DF) - (i >> 1)
            y = lax.bitcast_convert_type(i, jnp.float32)
            xh = x * jnp.float32(0.5)
            y = y * (jnp.float32(1.5) - xh * (y * y))
            for c in range(NCH):
                buf[r, pl.ds(c * LANES, LANES)] = (
                    (e[c] - mean_v) * (y * g_regs[c]) + b_regs[c])

        def row(r, carry):
            for uu in range(UNROLL):
                one_row(base + UNROLL * r + uu)
            return carry
        lax.fori_loop(0, HALF // UNROLL, row, 0)

    start_gather(0, 0)
    start_gather(1, 1)
    pltpu.sync_copy(pos_hbm.at[pl.ds(0, S)], pos_v)
    pltpu.sync_copy(g_hbm, gv)
    pltpu.sync_copy(b_hbm, bv)

    def process(u, k, wait_prev, prefetch, guard_prefetch):
        wait_gather_half(u, k, 0)
        compute(bufs[k], 0)
        start_scatter_half(u, k, 0)
        kn = (k + 2) % NBUF
        if wait_prev:
            wait_scatter(u - 1, kn)
        if prefetch:
            if guard_prefetch:
                @pl.when(u + 2 < UNITS_PER_W)
                def _():
                    start_gather(u + 2, kn)
            else:
                start_gather(u + 2, kn)
        wait_gather_half(u, k, 1)
        compute(bufs[k], 1)
        start_scatter_half(u, k, 1)

    process(0, 0, wait_prev=False, prefetch=True, guard_prefetch=False)
    process(1, 1, wait_prev=True, prefetch=True, guard_prefetch=False)

    def outer(g, carry):
        for j in range(NBUF):
            u = 2 + NBUF * g + j
            process(u, (2 + j) % NBUF, wait_prev=True, prefetch=True,
                    guard_prefetch=True)
        return carry
    lax.fori_loop(0, (UNITS_PER_W - 2) // NBUF, outer, 0)

    wait_scatter(UNITS_PER_W - 1, (UNITS_PER_W - 1) % NBUF)


def kernel(indices, token_table, pos_table, gamma, beta):
    idx2 = indices.astype(jnp.int32).reshape(2 * B, HALF)
    out = _bert_embed_sc(idx2, token_table, pos_table, gamma, beta)
    return out.reshape(B, S, H)
